# baseline (speedup 1.0000x reference)
import functools

import jax
import jax.numpy as jnp
from jax import lax
from jax.experimental import pallas as pl
from jax.experimental.pallas import tpu as pltpu
from jax.experimental.pallas import tpu_sc as plsc

N = 100000
D = 128
G = 512
L = 16
NW = 32

SC_C = 128
SC_PER_W = 9
SC_ROWS = NW * SC_PER_W * SC_C

V_C = 96
V_BASE = SC_ROWS
V_FULL = (N - V_BASE) // V_C
V_PER_W = 21
TAIL_BASE = V_BASE + V_FULL * V_C
TAIL = N - TAIL_BASE

PAIRS = (V_PER_W + 1) // 2
ROWS_PER_TILE = G // 16

_mesh = plsc.VectorSubcoreMesh(core_axis_name="c", subcore_axis_name="s")

_scratch = (
    [pltpu.VMEM((SC_C, D), jnp.float32) for _ in range(2)]
    + [pltpu.VMEM((SC_C,), jnp.int32) for _ in range(2)]
    + [pltpu.VMEM((V_C, D), jnp.float32) for _ in range(2)]
    + [pltpu.VMEM((V_C,), jnp.int32) for _ in range(2)]
    + [pltpu.VMEM((G, D), jnp.float32),
       pltpu.VMEM((SC_C,), jnp.int32)]
    + [pltpu.VMEM_SHARED((G, D), jnp.float32)]
    + [pltpu.SemaphoreType.DMA for _ in range(11)]
)


@functools.partial(
    pl.kernel,
    out_type=jax.ShapeDtypeStruct((2, G, D), jnp.float32),
    mesh=_mesh,
    scratch_types=_scratch,
)
def _sc_segment_sum(x_hbm, b_hbm, out_hbm, *refs):
    (sr0, sr1, si0, si1, vr0, vr1, vi0, vi1, lacc_v, ident_v, acc_sh,
     srsem0, srsem1, sisem0, sisem1, vrsem0, vrsem1, visem0, visem1,
     ssem0, ssem1, dsem) = refs
    s_rows = (sr0, sr1)
    s_ids = (si0, si1)
    s_rsem = (srsem0, srsem1)
    s_isem = (sisem0, sisem1)
    s_ssem = (ssem0, ssem1)
    v_rows = (vr0, vr1)
    v_ids = (vi0, vi1)
    v_rsem = (vrsem0, vrsem1)
    v_isem = (visem0, visem1)

    cid = lax.axis_index("c")
    sid = lax.axis_index("s")
    wid = cid * 16 + sid
    sg0 = wid * SC_PER_W
    vg0 = wid * V_PER_W

    def s_load(j, X):
        base = (sg0 + j) * SC_C
        pltpu.async_copy(b_hbm.at[pl.ds(base, SC_C)], s_ids[X], s_isem[X])
        pltpu.async_copy(x_hbm.at[pl.ds(base, SC_C)], s_rows[X], s_rsem[X])

    def s_wait_load(j, X):
        base = (sg0 + j) * SC_C
        pltpu.make_async_copy(b_hbm.at[pl.ds(base, SC_C)], s_ids[X],
                              s_isem[X]).wait()
        pltpu.make_async_copy(x_hbm.at[pl.ds(base, SC_C)], s_rows[X],
                              s_rsem[X]).wait()

    def v_valid(j):
        return vg0 + j < V_FULL

    def v_load(j, X):
        @pl.when((j < V_PER_W) & v_valid(j))
        def _():
            base = V_BASE + (vg0 + j) * V_C
            pltpu.async_copy(b_hbm.at[pl.ds(base, V_C)], v_ids[X],
                             v_isem[X])
            pltpu.async_copy(x_hbm.at[pl.ds(base, V_C)], v_rows[X],
                             v_rsem[X])

    lane = lax.iota(jnp.int32, L)
    zv = jnp.zeros((L,), jnp.float32)

    def reduce_rows(rows_ref, ids_ref, nrows):
        def group(gi, carry):
            idv = ids_ref[pl.ds(gi * L, L)]
            first = idv[0]
            last = idv[L - 1]

            @pl.when(first == last)
            def _():
                accs = [rows_ref[gi * L, pl.ds(k * L, L)]
                        for k in range(D // L)]
                for u in range(1, L):
                    for k in range(D // L):
                        accs[k] = accs[k] + rows_ref[gi * L + u,
                                                     pl.ds(k * L, L)]
                for k in range(D // L):
                    plsc.addupdate(lacc_v.at[first, pl.ds(k * L, L)],
                                   accs[k])

            @pl.when(first != last)
            def _():
                for u in range(L):
                    rid = idv[u]
                    for k in range(D // L):
                        v = rows_ref[gi * L + u, pl.ds(k * L, L)]
                        plsc.addupdate(lacc_v.at[rid, pl.ds(k * L, L)], v)

            return carry

        lax.fori_loop(0, nrows // L, group, 0)

    def v_process(j, X):
        @pl.when((j < V_PER_W) & v_valid(j))
        def _():
            base = V_BASE + (vg0 + j) * V_C
            pltpu.make_async_copy(b_hbm.at[pl.ds(base, V_C)], v_ids[X],
                                  v_isem[X]).wait()
            pltpu.make_async_copy(x_hbm.at[pl.ds(base, V_C)], v_rows[X],
                                  v_rsem[X]).wait()
            reduce_rows(v_rows[X], v_ids[X], V_C)

    s_load(0, 0)
    v_load(0, 0)
    v_load(1, 1)

    def zero_body(j, carry):
        for k in range(D // L):
            lacc_v[j, pl.ds(k * L, L)] = zv
        return carry

    lax.fori_loop(0, G, zero_body, 0)
    for k in range(SC_C // L):
        ident_v[pl.ds(k * L, L)] = lane + (k * L)
    pltpu.sync_copy(lacc_v.at[pl.ds(0, ROWS_PER_TILE)],
                    acc_sh.at[pl.ds(sid * ROWS_PER_TILE, ROWS_PER_TILE)])
    plsc.subcore_barrier()

    def s_drain(j, X):
        @pl.when((j >= 0) & (j < SC_PER_W))
        def _():
            pltpu.make_async_copy(s_rows[X], acc_sh.at[s_ids[X]],
                                  s_ssem[X]).wait()

    def s_reload(j, X):
        @pl.when(j < SC_PER_W)
        def _():
            s_load(j, X)

    def s_fire(j, X):
        @pl.when(j < SC_PER_W)
        def _():
            s_wait_load(j, X)
            pltpu.async_copy(s_rows[X], acc_sh.at[s_ids[X]],
                             s_ssem[X], add=True)

    def quad_body(q, carry):
        s_drain(2 * q - 1, 1)
        s_reload(2 * q + 1, 1)
        s_fire(2 * q, 0)
        v_process(4 * q, 0)
        v_load(4 * q + 2, 0)
        v_process(4 * q + 1, 1)
        v_load(4 * q + 3, 1)
        s_drain(2 * q, 0)
        s_reload(2 * q + 2, 0)
        s_fire(2 * q + 1, 1)
        v_process(4 * q + 2, 0)
        v_load(4 * q + 4, 0)
        v_process(4 * q + 3, 1)
        v_load(4 * q + 5, 1)
        return carry

    lax.fori_loop(0, (V_PER_W + 3) // 4, quad_body, 0)

    @pl.when(wid == NW - 1)
    def _():
        pltpu.sync_copy(b_hbm.at[pl.ds(TAIL_BASE, TAIL)],
                        v_ids[0].at[pl.ds(0, TAIL)])
        pltpu.sync_copy(x_hbm.at[pl.ds(TAIL_BASE, TAIL)],
                        v_rows[0].at[pl.ds(0, TAIL)])
        reduce_rows(v_rows[0], v_ids[0], TAIL)

    for q in range(G // SC_C):
        pltpu.async_copy(
            lacc_v.at[pl.ds(q * SC_C, SC_C)],
            acc_sh.at[pl.ds(q * SC_C, SC_C)].at[ident_v], dsem, add=True)
    for q in range(G // SC_C):
        pltpu.make_async_copy(
            lacc_v.at[pl.ds(q * SC_C, SC_C)],
            acc_sh.at[pl.ds(q * SC_C, SC_C)].at[ident_v], dsem).wait()

    plsc.subcore_barrier()

    pltpu.sync_copy(
        acc_sh.at[pl.ds(sid * ROWS_PER_TILE, ROWS_PER_TILE)],
        out_hbm.at[cid, pl.ds(sid * ROWS_PER_TILE, ROWS_PER_TILE)])


def _combine_body(p_ref, o_ref):
    o_ref[...] = p_ref[0] + p_ref[1]


_combine = pl.pallas_call(
    _combine_body,
    out_shape=jax.ShapeDtypeStruct((G, D), jnp.float32),
)


def kernel(input, batch, num_graphs):
    partials = _sc_segment_sum(input, batch.astype(jnp.int32))
    return _combine(partials)


# speedup vs baseline: 1.1822x; 1.1822x over previous
import functools

import jax
import jax.numpy as jnp
from jax import lax
from jax.experimental import pallas as pl
from jax.experimental.pallas import tpu as pltpu
from jax.experimental.pallas import tpu_sc as plsc

N = 100000
D = 128
G = 512

C = 128
FULL_CHUNKS = N // C
TAIL = N - FULL_CHUNKS * C
NW = 32
NBUF = 5
ROUNDS = 5
CPW = NBUF * ROUNDS
ROWS_PER_TILE = G // 16

_mesh = plsc.VectorSubcoreMesh(core_axis_name="c", subcore_axis_name="s")

_scratch = (
    [pltpu.VMEM((C, D), jnp.float32) for _ in range(NBUF)]
    + [pltpu.VMEM((C,), jnp.int32) for _ in range(NBUF)]
    + [pltpu.VMEM((TAIL,), jnp.int32),
       pltpu.VMEM((TAIL, D), jnp.float32),
       pltpu.VMEM((ROWS_PER_TILE, D), jnp.float32),
       pltpu.VMEM_SHARED((G, D), jnp.float32)]
    + [pltpu.SemaphoreType.DMA for _ in range(3 * NBUF)]
)


@functools.partial(
    pl.kernel,
    out_type=jax.ShapeDtypeStruct((2, G, D), jnp.float32),
    mesh=_mesh,
    scratch_types=_scratch,
)
def _sc_segment_sum(x_hbm, b_hbm, out_hbm, *refs):
    r_v = refs[0:NBUF]
    i_v = refs[NBUF:2 * NBUF]
    tidx_v, trows_v, z_v, acc_sh = refs[2 * NBUF:2 * NBUF + 4]
    rsem = refs[2 * NBUF + 4:2 * NBUF + 4 + NBUF]
    isem = refs[2 * NBUF + 4 + NBUF:2 * NBUF + 4 + 2 * NBUF]
    ssem = refs[2 * NBUF + 4 + 2 * NBUF:]

    cid = lax.axis_index("c")
    sid = lax.axis_index("s")
    wid = cid * 16 + sid
    g0 = wid * CPW

    def valid(c):
        return g0 + c < FULL_CHUNKS

    def load(c, b):
        @pl.when(valid(c))
        def _():
            base = (g0 + c) * C
            pltpu.async_copy(b_hbm.at[pl.ds(base, C)], i_v[b], isem[b])
            pltpu.async_copy(x_hbm.at[pl.ds(base, C)], r_v[b], rsem[b])

    def process(c, b):
        @pl.when(valid(c))
        def _():
            base = (g0 + c) * C
            pltpu.make_async_copy(b_hbm.at[pl.ds(base, C)], i_v[b],
                                  isem[b]).wait()
            pltpu.make_async_copy(x_hbm.at[pl.ds(base, C)], r_v[b],
                                  rsem[b]).wait()
            pltpu.async_copy(r_v[b], acc_sh.at[i_v[b]], ssem[b], add=True)

    def drain(c, b):
        @pl.when(valid(c))
        def _():
            pltpu.make_async_copy(r_v[b], acc_sh.at[i_v[b]], ssem[b]).wait()

    for b in range(NBUF):
        load(b, b)

    for j in range(ROWS_PER_TILE):
        for k in range(D // 16):
            z_v[j, pl.ds(k * 16, 16)] = jnp.zeros((16,), jnp.float32)
    pltpu.sync_copy(z_v, acc_sh.at[pl.ds(sid * ROWS_PER_TILE, ROWS_PER_TILE)])
    plsc.subcore_barrier()

    def round_body(r, carry):
        for b in range(NBUF):
            process(NBUF * r + b, b)
        for b in range(NBUF):
            @pl.when(r < ROUNDS - 1)
            def _():
                drain(NBUF * r + b, b)
                load(NBUF * (r + 1) + b, b)
        return carry

    lax.fori_loop(0, ROUNDS, round_body, 0)
    for b in range(NBUF):
        drain(NBUF * (ROUNDS - 1) + b, b)

    @pl.when(wid == NW - 1)
    def _():
        tbase = FULL_CHUNKS * C
        pltpu.sync_copy(b_hbm.at[pl.ds(tbase, TAIL)], tidx_v)
        pltpu.sync_copy(x_hbm.at[pl.ds(tbase, TAIL)], trows_v)
        pltpu.sync_copy(trows_v, acc_sh.at[tidx_v], add=True)

    plsc.subcore_barrier()

    pltpu.sync_copy(
        acc_sh.at[pl.ds(sid * ROWS_PER_TILE, ROWS_PER_TILE)],
        out_hbm.at[cid, pl.ds(sid * ROWS_PER_TILE, ROWS_PER_TILE)])


def _combine_body(p_ref, o_ref):
    o_ref[...] = p_ref[0] + p_ref[1]


_combine = pl.pallas_call(
    _combine_body,
    out_shape=jax.ShapeDtypeStruct((G, D), jnp.float32),
)


def kernel(input, batch, num_graphs):
    partials = _sc_segment_sum(input, batch.astype(jnp.int32))
    return _combine(partials)
